# Initial kernel scaffold; baseline (speedup 1.0000x reference)
import jax
import jax.numpy as jnp
from jax.experimental import pallas as pl


def kernel(x, edge_index, W1l, b1l, W1r, W2l, b2l, W2r):
    raise NotImplementedError("write your pallas kernel here")


# speedup vs baseline: 5.8862x; 5.8862x over previous
import functools

import jax
import jax.numpy as jnp
from jax import lax
from jax.experimental import pallas as pl
from jax.experimental.pallas import tpu as pltpu
from jax.experimental.pallas import tpu_sc as plsc

N_NODES = 10000
N_EDGES = 320000
D_IN = 128
D_HID = 256
D_OUT = 64

NC = 2
NS = 16
NT = NC * NS
CH = 128
NCH = -(-N_EDGES // (NT * CH))
E_PAD = NT * NCH * CH
RPT = 640
ACC_ROWS = NS * RPT


def _make_sc_agg(D, with_cnt):
  mesh = plsc.VectorSubcoreMesh(core_axis_name="c", subcore_axis_name="s")
  out_type = [jax.ShapeDtypeStruct((NC, ACC_ROWS, D), jnp.float32)]
  scratch = [
      pltpu.VMEM((NCH, CH), jnp.int32),
      pltpu.VMEM((NCH, CH), jnp.int32),
      pltpu.VMEM((CH, D), jnp.float32),
      pltpu.VMEM_SHARED((ACC_ROWS, D), jnp.float32),
      pltpu.SemaphoreType.DMA,
  ]
  if with_cnt:
    out_type.append(jax.ShapeDtypeStruct((NC, ACC_ROWS), jnp.float32))
    scratch += [
        pltpu.VMEM((CH,), jnp.float32),
        pltpu.VMEM((RPT,), jnp.float32),
        pltpu.VMEM_SHARED((ACC_ROWS,), jnp.float32),
    ]

  n16 = D // 16

  def body(table, src_h, dst_h, *rest):
    zeros16 = jnp.zeros((16,), jnp.float32)
    ones16 = jnp.ones((16,), jnp.float32)
    if with_cnt:
      (out_a, out_c, src_v, dst_v, rows, acc, sem, ones_v, zc_v, cacc) = rest
    else:
      (out_a, src_v, dst_v, rows, acc, sem) = rest
    cid = lax.axis_index("c")
    sid = lax.axis_index("s")
    tid = cid * NS + sid
    base = sid * RPT

    pltpu.sync_copy(src_h.at[tid], src_v)
    pltpu.sync_copy(dst_h.at[tid], dst_v)

    def zrow(i, _):
      rows[i // n16, pl.ds((i % n16) * 16, 16)] = zeros16
      return 0
    lax.fori_loop(0, CH * n16, zrow, 0)
    for k in range(RPT // CH):
      pltpu.sync_copy(rows, acc.at[pl.ds(base + k * CH, CH)])
    if with_cnt:
      def zc(i, _):
        ones_v[pl.ds(i * 16, 16)] = ones16
        zc_v[pl.ds(i * 16, 16)] = zeros16
        return 0
      lax.fori_loop(0, CH // 16, zc, 0)
      def zc2(i, _):
        zc_v[pl.ds(i * 16, 16)] = zeros16
        return 0
      lax.fori_loop(0, RPT // 16, zc2, 0)
      pltpu.sync_copy(zc_v, cacc.at[pl.ds(base, RPT)])
    plsc.subcore_barrier()

    def step(j, _):
      pltpu.async_copy(table.at[src_v.at[j]], rows, sem).wait()
      pltpu.sync_copy(rows, acc.at[dst_v.at[j]], add=True)
      if with_cnt:
        pltpu.sync_copy(ones_v, cacc.at[dst_v.at[j]], add=True)
      return 0
    lax.fori_loop(0, NCH, step, 0)
    plsc.subcore_barrier()

    pltpu.sync_copy(acc.at[pl.ds(base, RPT)], out_a.at[cid, pl.ds(base, RPT)])
    if with_cnt:
      pltpu.sync_copy(cacc.at[pl.ds(base, RPT)],
                      out_c.at[cid, pl.ds(base, RPT)])

  return pl.kernel(body, out_type=tuple(out_type), mesh=mesh,
                   scratch_types=tuple(scratch))


_sc_agg_l1 = _make_sc_agg(D_IN, True)
_sc_agg_l2 = _make_sc_agg(D_IN, False)

BR = 1000


def _dense_body(x, a0, a1, c0, c1, w1l, b1l, w1r, w2l, w2r,
                h_ref, p_ref, q_ref, ic_ref):
  c = jnp.maximum(c0[...] + c1[...], 1.0)
  mean = (a0[...] + a1[...]) / c
  h = lax.dot_general(mean, w1l[...], (((1,), (0,)), ((), ())),
                      preferred_element_type=jnp.float32)
  h += lax.dot_general(x[...], w1r[...], (((1,), (0,)), ((), ())),
                       preferred_element_type=jnp.float32)
  h = jnp.maximum(h + b1l[...], 0.0)
  h_ref[...] = h
  p_ref[:, :D_OUT] = lax.dot_general(h, w2l[...], (((1,), (0,)), ((), ())),
                                     preferred_element_type=jnp.float32)
  p_ref[:, D_OUT:] = jnp.zeros((BR, D_IN - D_OUT), jnp.float32)
  q_ref[...] = lax.dot_general(h, w2r[...], (((1,), (0,)), ((), ())),
                               preferred_element_type=jnp.float32)
  ic_ref[...] = 1.0 / c


def _final_body(g0, g1, ic, q, b2l, out_ref):
  g = g0[:, :D_OUT] + g1[:, :D_OUT]
  out_ref[...] = g * ic[...] + q[...] + b2l[...]


def _row_blk(d):
  return pl.BlockSpec((BR, d), lambda i: (i, 0))


def _full_blk(r, d):
  return pl.BlockSpec((r, d), lambda i: (0, 0))


_dense_call = pl.pallas_call(
    _dense_body,
    grid=(N_NODES // BR,),
    in_specs=[
        _row_blk(D_IN),
        _row_blk(D_IN),
        _row_blk(D_IN),
        _row_blk(1),
        _row_blk(1),
        _full_blk(D_IN, D_HID),
        _full_blk(1, D_HID),
        _full_blk(D_IN, D_HID),
        _full_blk(D_HID, D_OUT),
        _full_blk(D_HID, D_OUT),
    ],
    out_specs=[
        _row_blk(D_HID),
        _row_blk(D_IN),
        _row_blk(D_OUT),
        _row_blk(1),
    ],
    out_shape=[
        jax.ShapeDtypeStruct((N_NODES, D_HID), jnp.float32),
        jax.ShapeDtypeStruct((N_NODES, D_IN), jnp.float32),
        jax.ShapeDtypeStruct((N_NODES, D_OUT), jnp.float32),
        jax.ShapeDtypeStruct((N_NODES, 1), jnp.float32),
    ],
)

_final_call = pl.pallas_call(
    _final_body,
    grid=(N_NODES // BR,),
    in_specs=[
        _row_blk(D_IN),
        _row_blk(D_IN),
        _row_blk(1),
        _row_blk(D_OUT),
        _full_blk(1, D_OUT),
    ],
    out_specs=_row_blk(D_OUT),
    out_shape=jax.ShapeDtypeStruct((N_NODES, D_OUT), jnp.float32),
)


@jax.jit
def _run(x, edge_index, W1l, b1l, W1r, W2l, b2l, W2r):
  src = edge_index[0].astype(jnp.int32)
  dst = edge_index[1].astype(jnp.int32)
  pad = E_PAD - N_EDGES
  src = jnp.concatenate([src, jnp.zeros((pad,), jnp.int32)])
  dst = jnp.concatenate([dst, jnp.full((pad,), N_NODES, jnp.int32)])
  src_r = src.reshape(NT, NCH, CH)
  dst_r = dst.reshape(NT, NCH, CH)

  a, cnt = _sc_agg_l1(x, src_r, dst_r)
  c2 = cnt[:, :N_NODES, None]
  h, p, q, ic = _dense_call(x, a[0, :N_NODES], a[1, :N_NODES],
                            c2[0], c2[1], W1l.T, b1l[None, :], W1r.T,
                            W2l.T, W2r.T)
  (g,) = _sc_agg_l2(p, src_r, dst_r)
  return _final_call(g[0, :N_NODES], g[1, :N_NODES], ic, q, b2l[None, :])


def kernel(x, edge_index, W1l, b1l, W1r, W2l, b2l, W2r):
  return _run(x, edge_index, W1l, b1l, W1r, W2l, b2l, W2r)
